# Initial kernel scaffold; baseline (speedup 1.0000x reference)
import jax
import jax.numpy as jnp
from jax.experimental import pallas as pl


def kernel(x, table):
    raise NotImplementedError("write your pallas kernel here")


# speedup vs baseline: 2.6462x; 2.6462x over previous
import functools
import math

import jax
import jax.numpy as jnp
from jax import lax
from jax.experimental import pallas as pl
from jax.experimental.pallas import tpu as pltpu
from jax.experimental.pallas import tpu_sc as plsc

_VOCAB = 100000
_DIM = 128
_SCALE = math.sqrt(_DIM)

_NC = 2
_NS = 16
_NW = _NC * _NS

_GATHER = 128
_CHUNK = 640
_NSUB = _CHUNK // _GATHER


@functools.partial(jax.jit, static_argnames=("total",))
def _lookup(idx, table, *, total):
    per_w = total // _NW
    n_chunks = per_w // _CHUNK
    mesh = plsc.VectorSubcoreMesh(core_axis_name="c", subcore_axis_name="s")

    @functools.partial(
        pl.kernel,
        mesh=mesh,
        out_type=jax.ShapeDtypeStruct((total, _DIM), jnp.float32),
        scratch_types=[
            pltpu.VMEM((_CHUNK,), jnp.int32),
            pltpu.VMEM((_CHUNK, _DIM), jnp.float32),
            pltpu.SemaphoreType.DMA,
        ],
    )
    def k(idx_hbm, table_hbm, out_hbm, idx_v, rows_v, sem):
        wid = lax.axis_index("s") * _NC + lax.axis_index("c")
        base = wid * per_w

        def chunk_body(c, carry):
            off = base + c * _CHUNK
            pltpu.sync_copy(
                idx_hbm.at[pl.ds(off, _CHUNK)],
                idx_v.at[...],
            )
            copies = [
                pltpu.async_copy(
                    table_hbm.at[idx_v.at[pl.ds(g * _GATHER, _GATHER)]],
                    rows_v.at[pl.ds(g * _GATHER, _GATHER)],
                    sem,
                )
                for g in range(_NSUB)
            ]
            for cp in copies:
                cp.wait()

            def scale_row(i, carry2):
                for j in range(_DIM // 16):
                    sl = (i, pl.ds(j * 16, 16))
                    rows_v[sl] = rows_v[sl] * _SCALE
                return carry2

            lax.fori_loop(0, _CHUNK, scale_row, 0)
            pltpu.sync_copy(rows_v.at[...], out_hbm.at[pl.ds(off, _CHUNK)])
            return carry

        lax.fori_loop(0, n_chunks, chunk_body, 0)

    return k(idx, table)


def kernel(x, table):
    b, s = x.shape
    total = b * s
    idx = x.reshape(total).astype(jnp.int32)
    out = _lookup(idx, table, total=total)
    return out.reshape(b, s, _DIM)


# speedup vs baseline: 2.9204x; 1.1036x over previous
import functools
import math

import jax
import jax.numpy as jnp
from jax import lax
from jax.experimental import pallas as pl
from jax.experimental.pallas import tpu as pltpu
from jax.experimental.pallas import tpu_sc as plsc

_VOCAB = 100000
_DIM = 128
_SCALE = math.sqrt(_DIM)

_NC = 2
_NS = 16
_NW = _NC * _NS

_GATHER = 128
_CHUNK = 256
_NSUB = _CHUNK // _GATHER
_NBUF = 3


@functools.partial(jax.jit, static_argnames=("total",))
def _lookup(idx, table, *, total):
    per_w = total // _NW
    n_chunks = per_w // _CHUNK
    mesh = plsc.VectorSubcoreMesh(core_axis_name="c", subcore_axis_name="s")

    @functools.partial(
        pl.kernel,
        mesh=mesh,
        out_type=jax.ShapeDtypeStruct((total, _DIM), jnp.float32),
        scratch_types=[
            pltpu.VMEM((per_w,), jnp.int32),
            pltpu.VMEM((_NBUF, _CHUNK, _DIM), jnp.float32),
            pltpu.SemaphoreType.DMA,
            pltpu.SemaphoreType.DMA,
            pltpu.SemaphoreType.DMA,
            pltpu.SemaphoreType.DMA,
            pltpu.SemaphoreType.DMA,
            pltpu.SemaphoreType.DMA,
        ],
    )
    def k(idx_hbm, table_hbm, out_hbm, idx_v, rows_v, g0, g1, g2, w0, w1, w2):
        gsem = [g0, g1, g2]
        wsem = [w0, w1, w2]
        wid = lax.axis_index("s") * _NC + lax.axis_index("c")
        base = wid * per_w

        pltpu.sync_copy(idx_hbm.at[pl.ds(base, per_w)], idx_v.at[...])

        def fire_gather(c):
            b = c % _NBUF
            return [
                pltpu.async_copy(
                    table_hbm.at[idx_v.at[pl.ds(c * _CHUNK + g * _GATHER, _GATHER)]],
                    rows_v.at[b, pl.ds(g * _GATHER, _GATHER)],
                    gsem[b],
                )
                for g in range(_NSUB)
            ]

        def scale(b):
            def scale_row(i, carry):
                for j in range(_DIM // 16):
                    sl = (b, i, pl.ds(j * 16, 16))
                    rows_v[sl] = rows_v[sl] * _SCALE
                return carry

            lax.fori_loop(0, _CHUNK, scale_row, 0)

        pending = {c: fire_gather(c) for c in range(min(2, n_chunks))}
        wb = {}
        for c in range(n_chunks):
            b = c % _NBUF
            for cp in pending.pop(c):
                cp.wait()
            scale(b)
            wb[c] = pltpu.async_copy(
                rows_v.at[b],
                out_hbm.at[pl.ds(base + c * _CHUNK, _CHUNK)],
                wsem[b],
            )
            nxt = c + 2
            if nxt < n_chunks:
                prev = c - 1
                if prev in wb:
                    wb.pop(prev).wait()
                pending[nxt] = fire_gather(nxt)
        for c in sorted(wb):
            wb.pop(c).wait()

    return k(idx, table)


def kernel(x, table):
    b, s = x.shape
    total = b * s
    idx = x.reshape(total).astype(jnp.int32)
    out = _lookup(idx, table, total=total)
    return out.reshape(b, s, _DIM)


# speedup vs baseline: 5.0760x; 1.7381x over previous
import functools
import math

import jax
import jax.numpy as jnp
from jax import lax
from jax.experimental import pallas as pl
from jax.experimental.pallas import tpu as pltpu
from jax.experimental.pallas import tpu_sc as plsc

_VOCAB = 100000
_DIM = 128
_SCALE = math.sqrt(_DIM)

_NC = 2
_NS = 16
_NW = _NC * _NS

_SPAD = 64
_ROWS = 4
_NBUF = 3


@functools.partial(jax.jit, static_argnames=("n", "s"))
def _lookup(idx, table, *, n, s):
    rows_per_w = n // _NW
    n_chunks = rows_per_w // _ROWS
    mesh = plsc.VectorSubcoreMesh(core_axis_name="c", subcore_axis_name="s")

    @functools.partial(
        pl.kernel,
        mesh=mesh,
        out_type=jax.ShapeDtypeStruct((n, s, _DIM), jnp.float32),
        scratch_types=[
            pltpu.VMEM((rows_per_w * _SPAD,), jnp.int32),
            pltpu.VMEM((_NBUF, _ROWS, s, _DIM), jnp.float32),
            pltpu.SemaphoreType.DMA,
            pltpu.SemaphoreType.DMA,
            pltpu.SemaphoreType.DMA,
            pltpu.SemaphoreType.DMA,
            pltpu.SemaphoreType.DMA,
            pltpu.SemaphoreType.DMA,
        ],
    )
    def k(idx_hbm, table_hbm, out_hbm, idx_v, rows_v, g0, g1, g2, w0, w1, w2):
        gsem = [g0, g1, g2]
        wsem = [w0, w1, w2]
        wid = lax.axis_index("s") * _NC + lax.axis_index("c")
        row_base = wid * rows_per_w

        pltpu.sync_copy(
            idx_hbm.at[pl.ds(row_base * _SPAD, rows_per_w * _SPAD)],
            idx_v.at[...],
        )

        def fire_gather(c):
            b = c % _NBUF
            return [
                pltpu.async_copy(
                    table_hbm.at[idx_v.at[pl.ds((c * _ROWS + r) * _SPAD, s)]],
                    rows_v.at[b, r],
                    gsem[b],
                )
                for r in range(_ROWS)
            ]

        def scale(b):
            def scale_row(i, carry):
                for j in range(_DIM // 16):
                    sl = (b, i // s, i % s, pl.ds(j * 16, 16))
                    rows_v[sl] = rows_v[sl] * _SCALE
                return carry

            lax.fori_loop(0, _ROWS * s, scale_row, 0)

        pending = {c: fire_gather(c) for c in range(min(2, n_chunks))}
        wb = {}
        for c in range(n_chunks):
            b = c % _NBUF
            for cp in pending.pop(c):
                cp.wait()
            scale(b)
            wb[c] = pltpu.async_copy(
                rows_v.at[b],
                out_hbm.at[pl.ds(row_base + c * _ROWS, _ROWS)],
                wsem[b],
            )
            nxt = c + 2
            if nxt < n_chunks:
                prev = c - 1
                if prev in wb:
                    wb.pop(prev).wait()
                pending[nxt] = fire_gather(nxt)
        for c in sorted(wb):
            wb.pop(c).wait()

    return k(idx, table)


def kernel(x, table):
    n, s = x.shape
    idx = jnp.pad(x.astype(jnp.int32), ((0, 0), (0, _SPAD - s))).reshape(n * _SPAD)
    return _lookup(idx, table, n=n, s=s)
